# baseline (speedup 1.0000x reference)
import jax
import jax.numpy as jnp
from jax.experimental import pallas as pl
from jax.experimental.pallas import tpu as pltpu

TIME_STEPS = 64
T_BLK = 1
NBUF = 4
NSTEP = TIME_STEPS // T_BLK


def _onehot_kernel(x_ref, c_ref, s_ref, out_ref, tsp_ref, buf_ref, sems):
    dist = s_ref[:][None, :] * jnp.abs(x_ref[:][:, None] - c_ref[:][None, :])
    tsp_ref[:] = jnp.clip(dist.astype(jnp.int32), 0, TIME_STEPS - 1)
    tsp = tsp_ref[:]
    for i in range(NSTEP):
        k = i % NBUF
        if i >= NBUF:
            pltpu.make_async_copy(
                buf_ref.at[k],
                out_ref.at[pl.ds((i - NBUF) * T_BLK, T_BLK)],
                sems.at[k],
            ).wait()
        for dt in range(T_BLK):
            buf_ref[k, dt] = (tsp == i * T_BLK + dt).astype(jnp.float32)
        pltpu.make_async_copy(
            buf_ref.at[k],
            out_ref.at[pl.ds(i * T_BLK, T_BLK)],
            sems.at[k],
        ).start()
    for k in range(NBUF):
        i = NSTEP - NBUF + k
        pltpu.make_async_copy(
            buf_ref.at[i % NBUF],
            out_ref.at[pl.ds(i * T_BLK, T_BLK)],
            sems.at[i % NBUF],
        ).wait()


def kernel(x, center, scaling):
    b = x.shape[0]
    n = center.shape[0]
    return pl.pallas_call(
        _onehot_kernel,
        in_specs=[
            pl.BlockSpec(memory_space=pltpu.VMEM),
            pl.BlockSpec(memory_space=pltpu.VMEM),
            pl.BlockSpec(memory_space=pltpu.VMEM),
        ],
        out_specs=pl.BlockSpec(memory_space=pl.ANY),
        out_shape=jax.ShapeDtypeStruct((TIME_STEPS, b, n), jnp.float32),
        scratch_shapes=[
            pltpu.VMEM((b, n), jnp.int32),
            pltpu.VMEM((NBUF, T_BLK, b, n), jnp.float32),
            pltpu.SemaphoreType.DMA((NBUF,)),
        ],
    )(x, center, scaling)
